# baseline (speedup 1.0000x reference)
import functools

import jax
import jax.numpy as jnp
from jax import lax
from jax.experimental import pallas as pl
from jax.experimental.pallas import tpu as pltpu
from jax.experimental.pallas import tpu_sc as plsc

B = 16384
D = 64
NE = 1000000
NR = 1000
NC = 2
NS = 16
NW = NC * NS
BPW = B // NW
DH = D // 2


def _body(hidx, ridx, tidx, ent1d, rel_t, oh, og, ot,
          hidx_v, ridx_v, tidx_v, rel_tab, eidx, out_v, gsem, wsem):
    wid = lax.axis_index("s") * NC + lax.axis_index("c")
    base = wid * BPW

    pltpu.sync_copy(hidx.at[pl.ds(base, BPW)], hidx_v)
    pltpu.sync_copy(ridx.at[pl.ds(base, BPW)], ridx_v)
    pltpu.sync_copy(tidx.at[pl.ds(base, BPW)], tidx_v)
    pltpu.sync_copy(rel_t, rel_tab)

    def rel_chunk(jc, carry):
        iv = ridx_v[pl.ds(jc * 16, 16)]
        for d in range(D):
            dv = jnp.full((16,), d, jnp.int32)
            out_v[d, pl.ds(jc * 16, 16)] = plsc.load_gather(rel_tab, [dv, iv])
        return carry
    lax.fori_loop(0, BPW // 16, rel_chunk, 0)
    pltpu.async_copy(out_v, og.at[:, pl.ds(base, BPW)], wsem).wait()

    def gather_entity(idx_v, out_hbm):
        for half in range(2):
            def dim_row(dd, carry):
                d = half * DH + dd
                def build(jc, carry2):
                    iv = idx_v[pl.ds(jc * 16, 16)]
                    eidx[dd, pl.ds(jc * 16, 16)] = iv + d * NE
                    return carry2
                lax.fori_loop(0, BPW // 16, build, 0)
                def fire(q, carry2):
                    sl = pl.ds(q * 128, 128)
                    pltpu.async_copy(ent1d.at[eidx.at[dd, sl]],
                                     out_v.at[d, sl], gsem)
                    return carry2
                lax.fori_loop(0, BPW // 128, fire, 0)
                return carry
            lax.fori_loop(0, DH, dim_row, 0)
            pltpu.make_async_copy(
                rel_t.at[pl.ds(0, DH), pl.ds(0, BPW)],
                out_v.at[pl.ds(half * DH, DH)], gsem).wait()
        pltpu.async_copy(out_v, out_hbm.at[:, pl.ds(base, BPW)], wsem).wait()

    gather_entity(hidx_v, oh)
    gather_entity(tidx_v, ot)


_mesh = plsc.VectorSubcoreMesh(core_axis_name="c", subcore_axis_name="s")

_gather = functools.partial(
    pl.kernel,
    out_type=(
        jax.ShapeDtypeStruct((D, B), jnp.float32),
        jax.ShapeDtypeStruct((D, B), jnp.float32),
        jax.ShapeDtypeStruct((D, B), jnp.float32),
    ),
    mesh=_mesh,
    compiler_params=pltpu.CompilerParams(needs_layout_passes=False),
    scratch_types=[
        pltpu.VMEM((BPW,), jnp.int32),
        pltpu.VMEM((BPW,), jnp.int32),
        pltpu.VMEM((BPW,), jnp.int32),
        pltpu.VMEM((D, NR), jnp.float32),
        pltpu.VMEM((DH, BPW), jnp.int32),
        pltpu.VMEM((D, BPW), jnp.float32),
        pltpu.SemaphoreType.DMA,
        pltpu.SemaphoreType.DMA,
    ],
)(_body)


@jax.jit
def kernel(head, relation, tail, entity_table, relation_table):
    h = head.astype(jnp.int32)
    r = relation.astype(jnp.int32)
    t = tail.astype(jnp.int32)
    ent1d = entity_table.T.reshape(-1)
    rel_t = relation_table.T
    oh, og, ot = _gather(h, r, t, ent1d, rel_t)
    return (oh.T, og.T, ot.T)


# speedup vs baseline: 9.6446x; 9.6446x over previous
import functools

import jax
import jax.numpy as jnp
from jax import lax
from jax.experimental import pallas as pl
from jax.experimental.pallas import tpu as pltpu
from jax.experimental.pallas import tpu_sc as plsc

B = 16384
D = 64
NE = 1000000
NR = 1000
NC = 2
NS = 16
NW = NC * NS
BPW = B // NW
ICH = 64
NCH = BPW // ICH


def _body(hidx, ridx, tidx, ent_tbl, rel_tbl, oh, og, ot,
          idx_v, grp_v, out_v, gsem, wsem):
    wid = lax.axis_index("s") * NC + lax.axis_index("c")
    base = wid * BPW
    ent8 = ent_tbl.reshape((NE // 8, 8, D))
    rel8 = rel_tbl.reshape((NR // 8, 8, D))

    def gather_one(idx_hbm, table8, out_hbm):
        def chunk(c, carry):
            cbase = base + c * ICH
            pltpu.sync_copy(idx_hbm.at[pl.ds(cbase, ICH)], idx_v)

            def fire(jc, carry2):
                gv = idx_v[pl.ds(jc * 16, 16)] >> 3
                for j in range(16):
                    pltpu.async_copy(table8.at[gv[j]],
                                     grp_v.at[jc * 16 + j], gsem)
                return carry2
            lax.fori_loop(0, ICH // 16, fire, 0)
            pltpu.make_async_copy(table8.at[pl.ds(0, ICH)], grp_v, gsem).wait()

            def extract(jc, carry2):
                iv = idx_v[pl.ds(jc * 16, 16)]
                jv = lax.iota(jnp.int32, 16) + jc * 16
                sub = iv & 7
                for d in range(D):
                    dv = jnp.full((16,), d, jnp.int32)
                    out_v[d, pl.ds(c * ICH + jc * 16, 16)] = (
                        plsc.load_gather(grp_v, [jv, sub, dv]))
                return carry2
            lax.fori_loop(0, ICH // 16, extract, 0)
            return carry
        lax.fori_loop(0, NCH, chunk, 0)

        pltpu.async_copy(out_v, out_hbm.at[:, pl.ds(base, BPW)], wsem).wait()

    gather_one(ridx, rel8, og)
    gather_one(hidx, ent8, oh)
    gather_one(tidx, ent8, ot)


_mesh = plsc.VectorSubcoreMesh(core_axis_name="c", subcore_axis_name="s")

_gather = functools.partial(
    pl.kernel,
    out_type=(
        jax.ShapeDtypeStruct((D, B), jnp.float32),
        jax.ShapeDtypeStruct((D, B), jnp.float32),
        jax.ShapeDtypeStruct((D, B), jnp.float32),
    ),
    mesh=_mesh,
    compiler_params=pltpu.CompilerParams(needs_layout_passes=False),
    scratch_types=[
        pltpu.VMEM((ICH,), jnp.int32),
        pltpu.VMEM((ICH, 8, D), jnp.float32),
        pltpu.VMEM((D, BPW), jnp.float32),
        pltpu.SemaphoreType.DMA,
        pltpu.SemaphoreType.DMA,
    ],
)(_body)


@jax.jit
def kernel(head, relation, tail, entity_table, relation_table):
    h = head.astype(jnp.int32)
    r = relation.astype(jnp.int32)
    t = tail.astype(jnp.int32)
    oh, og, ot = _gather(h, r, t, entity_table, relation_table)
    return (oh.T, og.T, ot.T)


# speedup vs baseline: 12.6849x; 1.3152x over previous
import functools

import jax
import jax.numpy as jnp
from jax import lax
from jax.experimental.layout import Format, Layout
from jax.experimental import pallas as pl
from jax.experimental.pallas import tpu as pltpu
from jax.experimental.pallas import tpu_sc as plsc

B = 16384
D = 64
NE = 1000000
NR = 1000
NC = 2
NS = 16
NW = NC * NS
BPW = B // NW
ICH = 64
NCH = BPW // ICH


def _body(hidx, ridx, tidx, ent_tbl, rel_tbl, oh, og, ot,
          idx_v, grp_v, out_v, gsem, wsem):
    wid = lax.axis_index("s") * NC + lax.axis_index("c")
    base = wid * BPW
    ent8 = ent_tbl.reshape((NE // 8, 8, D))
    rel8 = rel_tbl.reshape((NR // 8, 8, D))

    def gather_one(idx_hbm, table8, out_hbm):
        def chunk(c, carry):
            cbase = base + c * ICH
            pltpu.sync_copy(idx_hbm.at[pl.ds(cbase, ICH)], idx_v)

            def fire(jc, carry2):
                gv = idx_v[pl.ds(jc * 16, 16)] >> 3
                for j in range(16):
                    pltpu.async_copy(table8.at[gv[j]],
                                     grp_v.at[jc * 16 + j], gsem)
                return carry2
            lax.fori_loop(0, ICH // 16, fire, 0)
            pltpu.make_async_copy(table8.at[pl.ds(0, ICH)], grp_v, gsem).wait()

            def extract(jc, carry2):
                iv = idx_v[pl.ds(jc * 16, 16)]
                jv = lax.iota(jnp.int32, 16) + jc * 16
                sub = iv & 7
                for d in range(D):
                    dv = jnp.full((16,), d, jnp.int32)
                    out_v[d, pl.ds(c * ICH + jc * 16, 16)] = (
                        plsc.load_gather(grp_v, [jv, sub, dv]))
                return carry2
            lax.fori_loop(0, ICH // 16, extract, 0)
            return carry
        lax.fori_loop(0, NCH, chunk, 0)

        pltpu.async_copy(out_v, out_hbm.at[:, pl.ds(base, BPW)], wsem).wait()

    gather_one(ridx, rel8, og)
    gather_one(hidx, ent8, oh)
    gather_one(tidx, ent8, ot)


_mesh = plsc.VectorSubcoreMesh(core_axis_name="c", subcore_axis_name="s")

_gather = functools.partial(
    pl.kernel,
    out_type=(
        jax.ShapeDtypeStruct((D, B), jnp.float32),
        jax.ShapeDtypeStruct((D, B), jnp.float32),
        jax.ShapeDtypeStruct((D, B), jnp.float32),
    ),
    mesh=_mesh,
    compiler_params=pltpu.CompilerParams(needs_layout_passes=False),
    scratch_types=[
        pltpu.VMEM((ICH,), jnp.int32),
        pltpu.VMEM((ICH, 8, D), jnp.float32),
        pltpu.VMEM((D, BPW), jnp.float32),
        pltpu.SemaphoreType.DMA,
        pltpu.SemaphoreType.DMA,
    ],
)(_body)


@jax.jit
def kernel(head, relation, tail, entity_table, relation_table):
    h = head.astype(jnp.int32)
    r = relation.astype(jnp.int32)
    t = tail.astype(jnp.int32)
    ent_rm = lax.optimization_barrier(entity_table.T).T
    rel_rm = lax.optimization_barrier(relation_table.T).T
    oh, og, ot = _gather(h, r, t, ent_rm, rel_rm)
    return (oh.T, og.T, ot.T)


# speedup vs baseline: 15.6909x; 1.2370x over previous
import functools

import jax
import jax.numpy as jnp
from jax import lax
from jax.experimental import pallas as pl
from jax.experimental.pallas import tpu as pltpu
from jax.experimental.pallas import tpu_sc as plsc

B = 16384
D = 64
NE = 1000000
NR = 1000
NC = 2
NS = 16
NW = NC * NS
BPW = B // NW
ICH = 128
NCH = BPW // ICH


def _body(hidx, ridx, tidx, ent_tbl, rel_tbl, oh, og, ot,
          idx_v, buf_a, buf_b, out_v, sem_a, sem_b, wsem):
    wid = lax.axis_index("s") * NC + lax.axis_index("c")
    base = wid * BPW
    bufs = (buf_a, buf_b)
    sems = (sem_a, sem_b)

    def gather_one(idx_hbm, tbl, out_hbm):
        pltpu.sync_copy(idx_hbm.at[pl.ds(base, BPW)], idx_v)

        def fire(c, buf, sem):
            def wave(jc, carry):
                gv = idx_v[pl.ds(c * ICH + jc * 16, 16)]
                for j in range(16):
                    pltpu.async_copy(tbl.at[pl.ds(gv[j], 1)],
                                     buf.at[pl.ds(jc * 16 + j, 1)], sem)
                return carry
            lax.fori_loop(0, ICH // 16, wave, 0)

        def extract(c, buf):
            def chunk16(jc, carry):
                jv = lax.iota(jnp.int32, 16) + jc * 16
                def dgroup(d8, carry2):
                    for dd in range(8):
                        d = d8 * 8 + dd
                        dv = jnp.zeros((16,), jnp.int32) + d
                        out_v[d, pl.ds(c * ICH + jc * 16, 16)] = (
                            plsc.load_gather(buf, [jv, dv]))
                    return carry2
                lax.fori_loop(0, D // 8, dgroup, 0)
                return carry
            lax.fori_loop(0, ICH // 16, chunk16, 0)

        fire(0, bufs[0], sems[0])
        for c in range(NCH):
            if c + 1 < NCH:
                fire(c + 1, bufs[(c + 1) % 2], sems[(c + 1) % 2])
            pltpu.make_async_copy(tbl.at[pl.ds(0, ICH)],
                                  bufs[c % 2], sems[c % 2]).wait()
            extract(c, bufs[c % 2])

        pltpu.async_copy(out_v, out_hbm.at[:, pl.ds(base, BPW)], wsem).wait()

    gather_one(ridx, rel_tbl, og)
    gather_one(hidx, ent_tbl, oh)
    gather_one(tidx, ent_tbl, ot)


_mesh = plsc.VectorSubcoreMesh(core_axis_name="c", subcore_axis_name="s")

_gather = functools.partial(
    pl.kernel,
    out_type=(
        jax.ShapeDtypeStruct((D, B), jnp.float32),
        jax.ShapeDtypeStruct((D, B), jnp.float32),
        jax.ShapeDtypeStruct((D, B), jnp.float32),
    ),
    mesh=_mesh,
    compiler_params=pltpu.CompilerParams(needs_layout_passes=False),
    scratch_types=[
        pltpu.VMEM((BPW,), jnp.int32),
        pltpu.VMEM((ICH, D), jnp.float32),
        pltpu.VMEM((ICH, D), jnp.float32),
        pltpu.VMEM((D, BPW), jnp.float32),
        pltpu.SemaphoreType.DMA,
        pltpu.SemaphoreType.DMA,
        pltpu.SemaphoreType.DMA,
    ],
)(_body)


@jax.jit
def kernel(head, relation, tail, entity_table, relation_table):
    h = head.astype(jnp.int32)
    r = relation.astype(jnp.int32)
    t = tail.astype(jnp.int32)
    ent_rm = lax.optimization_barrier(entity_table.T).T
    rel_rm = lax.optimization_barrier(relation_table.T).T
    oh, og, ot = _gather(h, r, t, ent_rm, rel_rm)
    return (oh.T, og.T, ot.T)


# speedup vs baseline: 15.9212x; 1.0147x over previous
import functools

import jax
import jax.numpy as jnp
from jax import lax
from jax.experimental import pallas as pl
from jax.experimental.pallas import tpu as pltpu
from jax.experimental.pallas import tpu_sc as plsc

B = 16384
D = 64
NE = 1000000
NR = 1000
NC = 2
NS = 16
NW = NC * NS
BPW = B // NW
ICH = 256
NCH = BPW // ICH


def _body(hidx, ridx, tidx, ent_tbl, rel_tbl, oh, og, ot,
          idx_v, buf_a, buf_b, out_v, sem_a, sem_b, wsem):
    wid = lax.axis_index("s") * NC + lax.axis_index("c")
    base = wid * BPW
    bufs = (buf_a, buf_b)
    sems = (sem_a, sem_b)

    def gather_one(idx_hbm, tbl, out_hbm, prev_write):
        pltpu.sync_copy(idx_hbm.at[pl.ds(base, BPW)], idx_v)

        def fire(c, buf, sem):
            def wave(jc, carry):
                gv = idx_v[pl.ds(c * ICH + jc * 16, 16)]
                for j in range(16):
                    pltpu.async_copy(tbl.at[pl.ds(gv[j], 1)],
                                     buf.at[pl.ds(jc * 16 + j, 1)], sem)
                return carry
            lax.fori_loop(0, ICH // 16, wave, 0)

        def extract(c, buf):
            def chunk16(jc, carry):
                jv = lax.iota(jnp.int32, 16) + jc * 16
                def dgroup(d8, carry2):
                    for dd in range(8):
                        d = d8 * 8 + dd
                        dv = jnp.zeros((16,), jnp.int32) + d
                        out_v[d, pl.ds(c * ICH + jc * 16, 16)] = (
                            plsc.load_gather(buf, [jv, dv]))
                    return carry2
                lax.fori_loop(0, D // 8, dgroup, 0)
                return carry
            lax.fori_loop(0, ICH // 16, chunk16, 0)

        fire(0, bufs[0], sems[0])
        for c in range(NCH):
            if c + 1 < NCH:
                fire(c + 1, bufs[(c + 1) % 2], sems[(c + 1) % 2])
            pltpu.make_async_copy(tbl.at[pl.ds(0, ICH)],
                                  bufs[c % 2], sems[c % 2]).wait()
            if c == 0 and prev_write is not None:
                prev_write.wait()
            extract(c, bufs[c % 2])

        return pltpu.async_copy(out_v, out_hbm.at[:, pl.ds(base, BPW)], wsem)

    w = gather_one(ridx, rel_tbl, og, None)
    w = gather_one(hidx, ent_tbl, oh, w)
    gather_one(tidx, ent_tbl, ot, w).wait()


_mesh = plsc.VectorSubcoreMesh(core_axis_name="c", subcore_axis_name="s")

_gather = functools.partial(
    pl.kernel,
    out_type=(
        jax.ShapeDtypeStruct((D, B), jnp.float32),
        jax.ShapeDtypeStruct((D, B), jnp.float32),
        jax.ShapeDtypeStruct((D, B), jnp.float32),
    ),
    mesh=_mesh,
    compiler_params=pltpu.CompilerParams(needs_layout_passes=False),
    scratch_types=[
        pltpu.VMEM((BPW,), jnp.int32),
        pltpu.VMEM((ICH, D), jnp.float32),
        pltpu.VMEM((ICH, D), jnp.float32),
        pltpu.VMEM((D, BPW), jnp.float32),
        pltpu.SemaphoreType.DMA,
        pltpu.SemaphoreType.DMA,
        pltpu.SemaphoreType.DMA,
    ],
)(_body)


@jax.jit
def kernel(head, relation, tail, entity_table, relation_table):
    h = head.astype(jnp.int32)
    r = relation.astype(jnp.int32)
    t = tail.astype(jnp.int32)
    ent_rm = lax.optimization_barrier(entity_table.T).T
    rel_rm = lax.optimization_barrier(relation_table.T).T
    oh, og, ot = _gather(h, r, t, ent_rm, rel_rm)
    return (oh.T, og.T, ot.T)
